# baseline (speedup 1.0000x reference)
import jax
import jax.numpy as jnp
from jax.experimental import pallas as pl
from jax.experimental.pallas import tpu as pltpu

_K = 32
_LAMBDA = 0.1
_TEMP = 0.1
_R = 256


def _norm_body(f_ref, x_ref):
    f = f_ref[...]
    n = jnp.sqrt(jnp.sum(f * f, axis=1, keepdims=True))
    x_ref[...] = f / jnp.maximum(n, 1e-12)


def _phase1_body(x_ref, xb_ref, f16_ref, f16b_ref, sim_ref, a_ref,
                 d1g_ref, d1s_ref):
    i = pl.program_id(0)
    x = x_ref[...]
    B = x.shape[0]
    xb = xb_ref[...]
    sim = jax.lax.dot_general(
        xb, x, (((1,), (1,)), ((), ())), preferred_element_type=jnp.float32)
    sim_ref[...] = sim
    iota = jax.lax.broadcasted_iota(jnp.int32, (_R, B), 1)

    bits = jax.lax.bitcast_convert_type(sim, jnp.uint32)
    ukey = jnp.where(
        (bits >> 31) != 0, ~bits, bits | jnp.uint32(0x80000000))

    kk = _K + 1
    kkf = float(kk)

    def bit_round(t, prefix):
        cand = prefix | (jnp.uint32(1) << (jnp.uint32(31) - t.astype(jnp.uint32)))
        cnt = jnp.sum(jnp.where(ukey >= cand, 1.0, 0.0), axis=1,
                      keepdims=True)
        return jnp.where(cnt >= kkf, cand, prefix)

    t33 = jax.lax.fori_loop(
        0, 32, bit_round, jnp.zeros((_R, 1), jnp.uint32))

    eq = ukey == t33
    gt_f = jnp.where(ukey > t33, 1.0, 0.0)
    cnt_gt = jnp.sum(gt_f, axis=1, keepdims=True)
    cnt_eq = jnp.sum(jnp.where(eq, 1.0, 0.0), axis=1, keepdims=True)
    need = kkf - cnt_gt

    def no_tie():
        return gt_f + jnp.where(eq, 1.0, 0.0)

    def tie_break():
        w = jnp.where(eq, B - iota, 0)

        def idx_round(t, prefix):
            cand = prefix | (jnp.int32(1) << (jnp.int32(12) - t))
            cnt = jnp.sum(jnp.where(w >= cand, 1.0, 0.0), axis=1,
                          keepdims=True)
            return jnp.where(cnt >= need, cand, prefix)

        wstar = jax.lax.fori_loop(
            0, 13, idx_round, jnp.zeros((_R, 1), jnp.int32))
        return gt_f + jnp.where(w >= wstar, 1.0, 0.0)

    sel = jax.lax.cond(jnp.all(cnt_eq == need), no_tie, tie_break)

    m = jnp.max(sim, axis=1, keepdims=True)
    i0 = jnp.min(jnp.where(sim == m, iota, B), axis=1, keepdims=True)
    af = jnp.where(iota == i0, 0.0, sel)
    a_ref[...] = af.astype(jnp.bfloat16)

    af16 = af.astype(jnp.bfloat16)
    d1g_ref[...] = jax.lax.dot_general(
        af16, f16_ref[...], (((1,), (0,)), ((), ())),
        preferred_element_type=jnp.float32)
    contrib = jax.lax.dot_general(
        af16, f16b_ref[...], (((0,), (0,)), ((), ())),
        preferred_element_type=jnp.float32)

    @pl.when(i == 0)
    def _():
        d1s_ref[...] = contrib

    @pl.when(i > 0)
    def _():
        d1s_ref[...] += contrib


def _mid1_body(d1g_ref, d1s_ref, v16_ref):
    v16_ref[...] = ((d1g_ref[...] + d1s_ref[...]) * (1.0 / _K)
                    ).astype(jnp.bfloat16)


def _phase2_body(a_ref, v16_ref, v16b_ref, gg_ref, gs_ref):
    i = pl.program_id(0)
    a16 = a_ref[...]
    gg_ref[...] = jax.lax.dot_general(
        a16, v16_ref[...], (((1,), (0,)), ((), ())),
        preferred_element_type=jnp.float32)
    contrib = jax.lax.dot_general(
        a16, v16b_ref[...], (((0,), (0,)), ((), ())),
        preferred_element_type=jnp.float32)

    @pl.when(i == 0)
    def _():
        gs_ref[...] = contrib

    @pl.when(i > 0)
    def _():
        gs_ref[...] += contrib


def _mid2_body(gg_ref, gs_ref, gn16_ref):
    g = (gg_ref[...] + gs_ref[...]) * (1.0 / _K)
    n = jnp.sqrt(jnp.sum(g * g, axis=1, keepdims=True))
    gn16_ref[...] = (g / jnp.maximum(n, 1e-8)).astype(jnp.bfloat16)


def _phase3_body(sim_ref, xb_ref, f16_ref, gn16_ref, out_ref):
    cos = jax.lax.dot_general(
        xb_ref[...].astype(jnp.bfloat16), gn16_ref[...],
        (((1,), (1,)), ((), ())), preferred_element_type=jnp.float32)
    logits = (sim_ref[...] + _LAMBDA * cos) / _TEMP
    m = jnp.max(logits, axis=1, keepdims=True)
    e = jnp.exp(logits - m)
    s = jnp.sum(e, axis=1, keepdims=True)
    acc = jax.lax.dot_general(
        e.astype(jnp.bfloat16), f16_ref[...], (((1,), (0,)), ((), ())),
        preferred_element_type=jnp.float32)
    out_ref[...] = acc / s


def kernel(features):
    B, D = features.shape
    nblk = B // _R
    f32 = jnp.float32
    bf16 = jnp.bfloat16

    f16 = features.astype(bf16)

    x = pl.pallas_call(
        _norm_body,
        in_specs=[pl.BlockSpec((B, D), lambda: (0, 0))],
        out_specs=pl.BlockSpec((B, D), lambda: (0, 0)),
        out_shape=jax.ShapeDtypeStruct((B, D), f32),
    )(features)

    sim, a, d1g, d1s = pl.pallas_call(
        _phase1_body,
        grid=(nblk,),
        in_specs=[
            pl.BlockSpec((B, D), lambda i: (0, 0)),
            pl.BlockSpec((_R, D), lambda i: (i, 0)),
            pl.BlockSpec((B, D), lambda i: (0, 0)),
            pl.BlockSpec((_R, D), lambda i: (i, 0)),
        ],
        out_specs=[
            pl.BlockSpec((_R, B), lambda i: (i, 0)),
            pl.BlockSpec((_R, B), lambda i: (i, 0)),
            pl.BlockSpec((_R, D), lambda i: (i, 0)),
            pl.BlockSpec((B, D), lambda i: (0, 0)),
        ],
        out_shape=[
            jax.ShapeDtypeStruct((B, B), f32),
            jax.ShapeDtypeStruct((B, B), bf16),
            jax.ShapeDtypeStruct((B, D), f32),
            jax.ShapeDtypeStruct((B, D), f32),
        ],
    )(x, x, f16, f16)

    v16 = pl.pallas_call(
        _mid1_body,
        in_specs=[
            pl.BlockSpec((B, D), lambda: (0, 0)),
            pl.BlockSpec((B, D), lambda: (0, 0)),
        ],
        out_specs=pl.BlockSpec((B, D), lambda: (0, 0)),
        out_shape=jax.ShapeDtypeStruct((B, D), bf16),
    )(d1g, d1s)

    gg, gs = pl.pallas_call(
        _phase2_body,
        grid=(nblk,),
        in_specs=[
            pl.BlockSpec((_R, B), lambda i: (i, 0)),
            pl.BlockSpec((B, D), lambda i: (0, 0)),
            pl.BlockSpec((_R, D), lambda i: (i, 0)),
        ],
        out_specs=[
            pl.BlockSpec((_R, D), lambda i: (i, 0)),
            pl.BlockSpec((B, D), lambda i: (0, 0)),
        ],
        out_shape=[
            jax.ShapeDtypeStruct((B, D), f32),
            jax.ShapeDtypeStruct((B, D), f32),
        ],
    )(a, v16, v16)

    gn16 = pl.pallas_call(
        _mid2_body,
        in_specs=[
            pl.BlockSpec((B, D), lambda: (0, 0)),
            pl.BlockSpec((B, D), lambda: (0, 0)),
        ],
        out_specs=pl.BlockSpec((B, D), lambda: (0, 0)),
        out_shape=jax.ShapeDtypeStruct((B, D), bf16),
    )(gg, gs)

    enhanced = pl.pallas_call(
        _phase3_body,
        grid=(nblk,),
        in_specs=[
            pl.BlockSpec((_R, B), lambda i: (i, 0)),
            pl.BlockSpec((_R, D), lambda i: (i, 0)),
            pl.BlockSpec((B, D), lambda i: (0, 0)),
            pl.BlockSpec((B, D), lambda i: (0, 0)),
        ],
        out_specs=pl.BlockSpec((_R, D), lambda i: (i, 0)),
        out_shape=jax.ShapeDtypeStruct((B, D), f32),
    )(sim, x, f16, gn16)

    return enhanced


# speedup vs baseline: 1.0125x; 1.0125x over previous
import jax
import jax.numpy as jnp
from jax.experimental import pallas as pl
from jax.experimental.pallas import tpu as pltpu

_K = 32
_LAMBDA = 0.1
_TEMP = 0.1
_R = 256


def _norm_body(f_ref, x_ref):
    f = f_ref[...]
    n = jnp.sqrt(jnp.sum(f * f, axis=1, keepdims=True))
    x_ref[...] = f / jnp.maximum(n, 1e-12)


def _phase1_body(x_ref, xb_ref, f_ref, fb_ref, sim_ref, a_ref,
                 d1g_ref, d1s_ref):
    i = pl.program_id(0)
    x = x_ref[...]
    B = x.shape[0]
    xb = xb_ref[...]
    sim = jax.lax.dot_general(
        xb, x, (((1,), (1,)), ((), ())), preferred_element_type=jnp.float32)
    sim_ref[...] = sim
    iota = jax.lax.broadcasted_iota(jnp.int32, (_R, B), 1)

    bits = jax.lax.bitcast_convert_type(sim, jnp.uint32)
    ukey = jnp.where(
        (bits >> 31) != 0, ~bits, bits | jnp.uint32(0x80000000))

    kk = _K + 1
    kkf = float(kk)

    def bit_round(t, prefix):
        cand = prefix | (jnp.uint32(1) << (jnp.uint32(31) - t.astype(jnp.uint32)))
        cnt = jnp.sum(jnp.where(ukey >= cand, 1.0, 0.0), axis=1,
                      keepdims=True)
        return jnp.where(cnt >= kkf, cand, prefix)

    t33 = jax.lax.fori_loop(
        0, 32, bit_round, jnp.zeros((_R, 1), jnp.uint32))

    eq = ukey == t33
    gt_f = jnp.where(ukey > t33, 1.0, 0.0)
    cnt_gt = jnp.sum(gt_f, axis=1, keepdims=True)
    cnt_eq = jnp.sum(jnp.where(eq, 1.0, 0.0), axis=1, keepdims=True)
    need = kkf - cnt_gt

    def no_tie():
        return gt_f + jnp.where(eq, 1.0, 0.0)

    def tie_break():
        w = jnp.where(eq, B - iota, 0)

        def idx_round(t, prefix):
            cand = prefix | (jnp.int32(1) << (jnp.int32(12) - t))
            cnt = jnp.sum(jnp.where(w >= cand, 1.0, 0.0), axis=1,
                          keepdims=True)
            return jnp.where(cnt >= need, cand, prefix)

        wstar = jax.lax.fori_loop(
            0, 13, idx_round, jnp.zeros((_R, 1), jnp.int32))
        return gt_f + jnp.where(w >= wstar, 1.0, 0.0)

    sel = jax.lax.cond(jnp.all(cnt_eq == need), no_tie, tie_break)

    m = jnp.max(sim, axis=1, keepdims=True)
    i0 = jnp.min(jnp.where(sim == m, iota, B), axis=1, keepdims=True)
    af = jnp.where(iota == i0, 0.0, sel)
    a_ref[...] = af.astype(jnp.bfloat16)

    d1g_ref[...] = jax.lax.dot_general(
        af, f_ref[...], (((1,), (0,)), ((), ())),
        preferred_element_type=jnp.float32)
    contrib = jax.lax.dot_general(
        af, fb_ref[...], (((0,), (0,)), ((), ())),
        preferred_element_type=jnp.float32)

    @pl.when(i == 0)
    def _():
        d1s_ref[...] = contrib

    @pl.when(i > 0)
    def _():
        d1s_ref[...] += contrib


def _mid1_body(d1g_ref, d1s_ref, v16_ref):
    v16_ref[...] = ((d1g_ref[...] + d1s_ref[...]) * (1.0 / _K)
                    ).astype(jnp.bfloat16)


def _phase2_body(a_ref, v16_ref, v16b_ref, gg_ref, gs_ref):
    i = pl.program_id(0)
    a16 = a_ref[...]
    gg_ref[...] = jax.lax.dot_general(
        a16, v16_ref[...], (((1,), (0,)), ((), ())),
        preferred_element_type=jnp.float32)
    contrib = jax.lax.dot_general(
        a16, v16b_ref[...], (((0,), (0,)), ((), ())),
        preferred_element_type=jnp.float32)

    @pl.when(i == 0)
    def _():
        gs_ref[...] = contrib

    @pl.when(i > 0)
    def _():
        gs_ref[...] += contrib


def _mid2_body(gg_ref, gs_ref, gn16_ref):
    g = (gg_ref[...] + gs_ref[...]) * (1.0 / _K)
    n = jnp.sqrt(jnp.sum(g * g, axis=1, keepdims=True))
    gn16_ref[...] = (g / jnp.maximum(n, 1e-8)).astype(jnp.bfloat16)


def _phase3_body(sim_ref, xb_ref, f16_ref, gn16_ref, out_ref):
    cos = jax.lax.dot_general(
        xb_ref[...].astype(jnp.bfloat16), gn16_ref[...],
        (((1,), (1,)), ((), ())), preferred_element_type=jnp.float32)
    logits = (sim_ref[...] + _LAMBDA * cos) / _TEMP
    m = jnp.max(logits, axis=1, keepdims=True)
    e = jnp.exp(logits - m)
    s = jnp.sum(e, axis=1, keepdims=True)
    acc = jax.lax.dot_general(
        e.astype(jnp.bfloat16), f16_ref[...], (((1,), (0,)), ((), ())),
        preferred_element_type=jnp.float32)
    out_ref[...] = acc / s


def kernel(features):
    B, D = features.shape
    nblk = B // _R
    f32 = jnp.float32
    bf16 = jnp.bfloat16

    x = pl.pallas_call(
        _norm_body,
        in_specs=[pl.BlockSpec((B, D), lambda: (0, 0))],
        out_specs=pl.BlockSpec((B, D), lambda: (0, 0)),
        out_shape=jax.ShapeDtypeStruct((B, D), f32),
    )(features)

    sim, a, d1g, d1s = pl.pallas_call(
        _phase1_body,
        grid=(nblk,),
        in_specs=[
            pl.BlockSpec((B, D), lambda i: (0, 0)),
            pl.BlockSpec((_R, D), lambda i: (i, 0)),
            pl.BlockSpec((B, D), lambda i: (0, 0)),
            pl.BlockSpec((_R, D), lambda i: (i, 0)),
        ],
        out_specs=[
            pl.BlockSpec((_R, B), lambda i: (i, 0)),
            pl.BlockSpec((_R, B), lambda i: (i, 0)),
            pl.BlockSpec((_R, D), lambda i: (i, 0)),
            pl.BlockSpec((B, D), lambda i: (0, 0)),
        ],
        out_shape=[
            jax.ShapeDtypeStruct((B, B), f32),
            jax.ShapeDtypeStruct((B, B), bf16),
            jax.ShapeDtypeStruct((B, D), f32),
            jax.ShapeDtypeStruct((B, D), f32),
        ],
    )(x, x, features, features)

    v16 = pl.pallas_call(
        _mid1_body,
        in_specs=[
            pl.BlockSpec((B, D), lambda: (0, 0)),
            pl.BlockSpec((B, D), lambda: (0, 0)),
        ],
        out_specs=pl.BlockSpec((B, D), lambda: (0, 0)),
        out_shape=jax.ShapeDtypeStruct((B, D), bf16),
    )(d1g, d1s)

    gg, gs = pl.pallas_call(
        _phase2_body,
        grid=(nblk,),
        in_specs=[
            pl.BlockSpec((_R, B), lambda i: (i, 0)),
            pl.BlockSpec((B, D), lambda i: (0, 0)),
            pl.BlockSpec((_R, D), lambda i: (i, 0)),
        ],
        out_specs=[
            pl.BlockSpec((_R, D), lambda i: (i, 0)),
            pl.BlockSpec((B, D), lambda i: (0, 0)),
        ],
        out_shape=[
            jax.ShapeDtypeStruct((B, D), f32),
            jax.ShapeDtypeStruct((B, D), f32),
        ],
    )(a, v16, v16)

    gn16 = pl.pallas_call(
        _mid2_body,
        in_specs=[
            pl.BlockSpec((B, D), lambda: (0, 0)),
            pl.BlockSpec((B, D), lambda: (0, 0)),
        ],
        out_specs=pl.BlockSpec((B, D), lambda: (0, 0)),
        out_shape=jax.ShapeDtypeStruct((B, D), bf16),
    )(gg, gs)

    enhanced = pl.pallas_call(
        _phase3_body,
        grid=(nblk,),
        in_specs=[
            pl.BlockSpec((_R, B), lambda i: (i, 0)),
            pl.BlockSpec((_R, D), lambda i: (i, 0)),
            pl.BlockSpec((B, D), lambda i: (0, 0)),
            pl.BlockSpec((B, D), lambda i: (0, 0)),
        ],
        out_specs=pl.BlockSpec((_R, D), lambda i: (i, 0)),
        out_shape=jax.ShapeDtypeStruct((B, D), f32),
    )(sim, x, features.astype(bf16), gn16)

    return enhanced


# speedup vs baseline: 1.0337x; 1.0209x over previous
import jax
import jax.numpy as jnp
from jax.experimental import pallas as pl
from jax.experimental.pallas import tpu as pltpu

_K = 32
_LAMBDA = 0.1
_TEMP = 0.1
_R = 256


def _norm_body(f_ref, x_ref):
    f = f_ref[...]
    n = jnp.sqrt(jnp.sum(f * f, axis=1, keepdims=True))
    x_ref[...] = f / jnp.maximum(n, 1e-12)


def _phase1_body(x_ref, xb_ref, f_ref, fb_ref, sim_ref, a_ref,
                 d1g_ref, d1s_ref):
    i = pl.program_id(0)
    x = x_ref[...]
    B = x.shape[0]
    xb = xb_ref[...]
    sim = jax.lax.dot_general(
        xb, x, (((1,), (1,)), ((), ())), preferred_element_type=jnp.float32)
    sim_ref[...] = sim
    iota = jax.lax.broadcasted_iota(jnp.int32, (_R, B), 1)

    bits = jax.lax.bitcast_convert_type(sim, jnp.uint32)
    ukey = jnp.where(
        (bits >> 31) != 0, ~bits, bits | jnp.uint32(0x80000000))

    kk = _K + 1
    kkf = float(kk)

    n_pos = jnp.sum(jnp.where(ukey >= jnp.uint32(0x80000000), 1.0, 0.0),
                    axis=1, keepdims=True)
    p0 = jnp.where(n_pos >= kkf, jnp.uint32(0x80000000),
                   jnp.uint32(0x40000000))

    def bit_round(t, prefix):
        cand = prefix | (jnp.uint32(1) << (jnp.uint32(29) - t.astype(jnp.uint32)))
        cnt = jnp.sum(jnp.where(ukey >= cand, 1.0, 0.0), axis=1,
                      keepdims=True)
        return jnp.where(cnt >= kkf, cand, prefix)

    t33 = jax.lax.fori_loop(0, 30, bit_round, p0)

    eq = ukey == t33
    gt_f = jnp.where(ukey > t33, 1.0, 0.0)
    cnt_gt = jnp.sum(gt_f, axis=1, keepdims=True)
    cnt_eq = jnp.sum(jnp.where(eq, 1.0, 0.0), axis=1, keepdims=True)
    need = kkf - cnt_gt

    def no_tie():
        return gt_f + jnp.where(eq, 1.0, 0.0)

    def tie_break():
        w = jnp.where(eq, B - iota, 0)

        def idx_round(t, prefix):
            cand = prefix | (jnp.int32(1) << (jnp.int32(12) - t))
            cnt = jnp.sum(jnp.where(w >= cand, 1.0, 0.0), axis=1,
                          keepdims=True)
            return jnp.where(cnt >= need, cand, prefix)

        wstar = jax.lax.fori_loop(
            0, 13, idx_round, jnp.zeros((_R, 1), jnp.int32))
        return gt_f + jnp.where(w >= wstar, 1.0, 0.0)

    sel = jax.lax.cond(jnp.all(cnt_eq == need), no_tie, tie_break)

    m = jnp.max(sim, axis=1, keepdims=True)
    i0 = jnp.min(jnp.where(sim == m, iota, B), axis=1, keepdims=True)
    af = jnp.where(iota == i0, 0.0, sel)
    a_ref[...] = af.astype(jnp.bfloat16)

    d1g_ref[...] = jax.lax.dot_general(
        af, f_ref[...], (((1,), (0,)), ((), ())),
        preferred_element_type=jnp.float32)
    contrib = jax.lax.dot_general(
        af, fb_ref[...], (((0,), (0,)), ((), ())),
        preferred_element_type=jnp.float32)

    @pl.when(i == 0)
    def _():
        d1s_ref[...] = contrib

    @pl.when(i > 0)
    def _():
        d1s_ref[...] += contrib


def _mid1_body(d1g_ref, d1s_ref, v16_ref):
    v16_ref[...] = ((d1g_ref[...] + d1s_ref[...]) * (1.0 / _K)
                    ).astype(jnp.bfloat16)


def _phase2_body(a_ref, v16_ref, v16b_ref, gg_ref, gs_ref):
    i = pl.program_id(0)
    a16 = a_ref[...]
    gg_ref[...] = jax.lax.dot_general(
        a16, v16_ref[...], (((1,), (0,)), ((), ())),
        preferred_element_type=jnp.float32)
    contrib = jax.lax.dot_general(
        a16, v16b_ref[...], (((0,), (0,)), ((), ())),
        preferred_element_type=jnp.float32)

    @pl.when(i == 0)
    def _():
        gs_ref[...] = contrib

    @pl.when(i > 0)
    def _():
        gs_ref[...] += contrib


def _mid2_body(gg_ref, gs_ref, gn16_ref):
    g = (gg_ref[...] + gs_ref[...]) * (1.0 / _K)
    n = jnp.sqrt(jnp.sum(g * g, axis=1, keepdims=True))
    gn16_ref[...] = (g / jnp.maximum(n, 1e-8)).astype(jnp.bfloat16)


def _phase3_body(sim_ref, xb_ref, f_ref, gn16_ref, out_ref):
    cos = jax.lax.dot_general(
        xb_ref[...].astype(jnp.bfloat16), gn16_ref[...],
        (((1,), (1,)), ((), ())), preferred_element_type=jnp.float32)
    logits = (sim_ref[...] + _LAMBDA * cos) / _TEMP
    m = jnp.max(logits, axis=1, keepdims=True)
    e = jnp.exp(logits - m)
    s = jnp.sum(e, axis=1, keepdims=True)
    acc = jax.lax.dot_general(
        e, f_ref[...], (((1,), (0,)), ((), ())),
        preferred_element_type=jnp.float32)
    out_ref[...] = acc / s


def kernel(features):
    B, D = features.shape
    nblk = B // _R
    f32 = jnp.float32
    bf16 = jnp.bfloat16

    x = pl.pallas_call(
        _norm_body,
        in_specs=[pl.BlockSpec((B, D), lambda: (0, 0))],
        out_specs=pl.BlockSpec((B, D), lambda: (0, 0)),
        out_shape=jax.ShapeDtypeStruct((B, D), f32),
    )(features)

    sim, a, d1g, d1s = pl.pallas_call(
        _phase1_body,
        grid=(nblk,),
        in_specs=[
            pl.BlockSpec((B, D), lambda i: (0, 0)),
            pl.BlockSpec((_R, D), lambda i: (i, 0)),
            pl.BlockSpec((B, D), lambda i: (0, 0)),
            pl.BlockSpec((_R, D), lambda i: (i, 0)),
        ],
        out_specs=[
            pl.BlockSpec((_R, B), lambda i: (i, 0)),
            pl.BlockSpec((_R, B), lambda i: (i, 0)),
            pl.BlockSpec((_R, D), lambda i: (i, 0)),
            pl.BlockSpec((B, D), lambda i: (0, 0)),
        ],
        out_shape=[
            jax.ShapeDtypeStruct((B, B), f32),
            jax.ShapeDtypeStruct((B, B), bf16),
            jax.ShapeDtypeStruct((B, D), f32),
            jax.ShapeDtypeStruct((B, D), f32),
        ],
    )(x, x, features, features)

    v16 = pl.pallas_call(
        _mid1_body,
        in_specs=[
            pl.BlockSpec((B, D), lambda: (0, 0)),
            pl.BlockSpec((B, D), lambda: (0, 0)),
        ],
        out_specs=pl.BlockSpec((B, D), lambda: (0, 0)),
        out_shape=jax.ShapeDtypeStruct((B, D), bf16),
    )(d1g, d1s)

    gg, gs = pl.pallas_call(
        _phase2_body,
        grid=(nblk,),
        in_specs=[
            pl.BlockSpec((_R, B), lambda i: (i, 0)),
            pl.BlockSpec((B, D), lambda i: (0, 0)),
            pl.BlockSpec((_R, D), lambda i: (i, 0)),
        ],
        out_specs=[
            pl.BlockSpec((_R, D), lambda i: (i, 0)),
            pl.BlockSpec((B, D), lambda i: (0, 0)),
        ],
        out_shape=[
            jax.ShapeDtypeStruct((B, D), f32),
            jax.ShapeDtypeStruct((B, D), f32),
        ],
    )(a, v16, v16)

    gn16 = pl.pallas_call(
        _mid2_body,
        in_specs=[
            pl.BlockSpec((B, D), lambda: (0, 0)),
            pl.BlockSpec((B, D), lambda: (0, 0)),
        ],
        out_specs=pl.BlockSpec((B, D), lambda: (0, 0)),
        out_shape=jax.ShapeDtypeStruct((B, D), bf16),
    )(gg, gs)

    enhanced = pl.pallas_call(
        _phase3_body,
        grid=(nblk,),
        in_specs=[
            pl.BlockSpec((_R, B), lambda i: (i, 0)),
            pl.BlockSpec((_R, D), lambda i: (i, 0)),
            pl.BlockSpec((B, D), lambda i: (0, 0)),
            pl.BlockSpec((B, D), lambda i: (0, 0)),
        ],
        out_specs=pl.BlockSpec((_R, D), lambda i: (i, 0)),
        out_shape=jax.ShapeDtypeStruct((B, D), f32),
    )(sim, x, features, gn16)

    return enhanced
